# Initial kernel scaffold; baseline (speedup 1.0000x reference)
import jax
import jax.numpy as jnp
from jax.experimental import pallas as pl


def kernel(x, edge_index, edge_attr, W_edge, b_edge, W1, b1, W2, b2):
    raise NotImplementedError("write your pallas kernel here")


# speedup vs baseline: 48.0471x; 48.0471x over previous
import functools

import jax
import jax.numpy as jnp
from jax import lax
from jax.experimental import pallas as pl
from jax.experimental.pallas import tpu as pltpu
from jax.experimental.pallas import tpu_sc as plsc

B, BANDS, N, E = 8, 9, 512, 16384
G = B * BANDS
IN_C, HID, OUT = 32, 64, 128
NW = 32
Q = 4
EQ = E // Q
TASKS = G * Q
TPW = TASKS // NW
L = 16
NC = N * IN_C


def _sc_scatter(x, src, dst, ea, params):
  mesh = plsc.VectorSubcoreMesh(core_axis_name="c", subcore_axis_name="s")

  @functools.partial(
      pl.kernel,
      mesh=mesh,
      out_type=jax.ShapeDtypeStruct((G, Q, NC), jnp.float32),
      scratch_types=[
          pltpu.VMEM((NC,), jnp.float32),
          pltpu.VMEM((NC,), jnp.float32),
          pltpu.VMEM((EQ,), jnp.int32),
          pltpu.VMEM((EQ,), jnp.int32),
          pltpu.VMEM((EQ,), jnp.float32),
          pltpu.VMEM((4 * L,), jnp.float32),
      ],
  )
  def k(x_hbm, src_hbm, dst_hbm, ea_hbm, par_hbm, out_hbm,
        x_v, aggr_v, src_v, dst_v, ea_v, par_v):
    cid = lax.axis_index("c")
    sid = lax.axis_index("s")
    wid = sid * 2 + cid
    pltpu.sync_copy(par_hbm, par_v)
    we0 = par_v[pl.ds(0, L)]
    we1 = par_v[pl.ds(L, L)]
    be0 = par_v[pl.ds(2 * L, L)]
    be1 = par_v[pl.ds(3 * L, L)]
    zero = jnp.zeros((L,), jnp.float32)

    def task(i, carry):
      t = wid * TPW + i
      g = t // Q
      q = t % Q
      pltpu.sync_copy(x_hbm.at[g], x_v)
      pltpu.sync_copy(src_hbm.at[g, pl.ds(q * EQ, EQ)], src_v)
      pltpu.sync_copy(dst_hbm.at[g, pl.ds(q * EQ, EQ)], dst_v)
      pltpu.sync_copy(ea_hbm.at[g, pl.ds(q * EQ, EQ)], ea_v)

      def zloop(n, c):
        aggr_v[pl.ds(n * L, L)] = zero
        return c

      lax.fori_loop(0, NC // L, zloop, 0, unroll=8)

      def eloop(j, c):
        base = j * L
        s16 = src_v[pl.ds(base, L)] * IN_C
        d16 = dst_v[pl.ds(base, L)] * IN_C
        a16 = ea_v[pl.ds(base, L)]
        for lane in range(L):
          s = s16[lane]
          d = d16[lane]
          a = a16[lane]
          m0 = jnp.maximum(x_v[pl.ds(s, L)] + (a * we0 + be0), 0.0)
          m1 = jnp.maximum(x_v[pl.ds(s + L, L)] + (a * we1 + be1), 0.0)
          plsc.addupdate(aggr_v.at[pl.ds(d, L)], m0)
          plsc.addupdate(aggr_v.at[pl.ds(d + L, L)], m1)
        return c

      lax.fori_loop(0, EQ // L, eloop, 0)

      pltpu.sync_copy(aggr_v, out_hbm.at[g, q])
      return carry

    lax.fori_loop(0, TPW, task, 0)

  return k(x, src, dst, ea, params)


GB = 8


def _mlp_body(x_ref, p_ref, w1_ref, b1_ref, w2_ref, b2_ref, o_ref):
  xb = x_ref[...]
  h = xb + p_ref[...].sum(axis=1)
  h2 = jnp.maximum(
      h.reshape(GB * N, IN_C) @ w1_ref[...] + b1_ref[...], 0.0)
  pooled = h2.reshape(GB, N, HID).sum(axis=1) * (1.0 / N)
  o_ref[...] = pooled @ w2_ref[...] + b2_ref[...]


def _mlp(x, partials, W1, b1, W2, b2):
  return pl.pallas_call(
      _mlp_body,
      out_shape=jax.ShapeDtypeStruct((G, OUT), jnp.float32),
      grid=(G // GB,),
      in_specs=[
          pl.BlockSpec((GB, N, IN_C), lambda i: (i, 0, 0)),
          pl.BlockSpec((GB, Q, N, IN_C), lambda i: (i, 0, 0, 0)),
          pl.BlockSpec((IN_C, HID), lambda i: (0, 0)),
          pl.BlockSpec((1, HID), lambda i: (0, 0)),
          pl.BlockSpec((HID, OUT), lambda i: (0, 0)),
          pl.BlockSpec((1, OUT), lambda i: (0, 0)),
      ],
      out_specs=pl.BlockSpec((GB, OUT), lambda i: (i, 0)),
  )(x, partials, W1, b1, W2, b2)


def kernel(x, edge_index, edge_attr, W_edge, b_edge, W1, b1, W2, b2):
  x3 = x.reshape(G, N, IN_C)
  ei = edge_index.reshape(G, 2, E).astype(jnp.int32)
  src = ei[:, 0, :]
  dst = ei[:, 1, :]
  ea = edge_attr.reshape(G, E)
  params = jnp.concatenate(
      [W_edge.reshape(-1), b_edge.reshape(-1)]).astype(jnp.float32)
  partials = _sc_scatter(x3.reshape(G, NC), src, dst, ea, params)
  tokens = _mlp(x3, partials.reshape(G, Q, N, IN_C), W1,
                b1.reshape(1, HID), W2, b2.reshape(1, OUT))
  return tokens.reshape(B, BANDS, OUT)


# speedup vs baseline: 67.6601x; 1.4082x over previous
import functools

import jax
import jax.numpy as jnp
from jax import lax
from jax.experimental import pallas as pl
from jax.experimental.pallas import tpu as pltpu
from jax.experimental.pallas import tpu_sc as plsc

B, BANDS, N, E = 8, 9, 512, 16384
G = B * BANDS
IN_C, HID, OUT = 32, 64, 128
NW = 32
Q = 4
EQ = E // Q
TASKS = G * Q
TPW = TASKS // NW
L = 16
NC = N * IN_C


def _sc_scatter(x, src, dst, ea, params):
  mesh = plsc.VectorSubcoreMesh(core_axis_name="c", subcore_axis_name="s")

  @functools.partial(
      pl.kernel,
      mesh=mesh,
      out_type=jax.ShapeDtypeStruct((G, Q, NC), jnp.float32),
      scratch_types=[
          pltpu.VMEM((NC,), jnp.float32),
          pltpu.VMEM((NC,), jnp.float32),
          pltpu.VMEM((EQ,), jnp.int32),
          pltpu.VMEM((EQ,), jnp.int32),
          pltpu.VMEM((EQ,), jnp.float32),
          pltpu.VMEM((4 * L,), jnp.float32),
      ],
  )
  def k(x_hbm, src_hbm, dst_hbm, ea_hbm, par_hbm, out_hbm,
        x_v, aggr_v, src_v, dst_v, ea_v, par_v):
    cid = lax.axis_index("c")
    sid = lax.axis_index("s")
    wid = sid * 2 + cid
    pltpu.sync_copy(par_hbm, par_v)
    we0 = par_v[pl.ds(0, L)]
    we1 = par_v[pl.ds(L, L)]
    be0 = par_v[pl.ds(2 * L, L)]
    be1 = par_v[pl.ds(3 * L, L)]
    zero = jnp.zeros((L,), jnp.float32)

    def task(i, carry):
      t = wid * TPW + i
      g = t // Q
      q = t % Q
      pltpu.sync_copy(x_hbm.at[g], x_v)
      pltpu.sync_copy(src_hbm.at[g, pl.ds(q * EQ, EQ)], src_v)
      pltpu.sync_copy(dst_hbm.at[g, pl.ds(q * EQ, EQ)], dst_v)
      pltpu.sync_copy(ea_hbm.at[g, pl.ds(q * EQ, EQ)], ea_v)

      @plsc.parallel_loop(0, NC // L, unroll=8)
      def zloop(n):
        aggr_v[pl.ds(n * L, L)] = zero

      @plsc.parallel_loop(0, EQ // L, unroll=2)
      def eloop(j):
        base = j * L
        s16 = src_v[pl.ds(base, L)] * IN_C
        d16 = dst_v[pl.ds(base, L)] * IN_C
        a16 = ea_v[pl.ds(base, L)]
        for lane in range(L):
          s = s16[lane]
          d = d16[lane]
          a = a16[lane]
          m0 = jnp.maximum(x_v[pl.ds(s, L)] + (a * we0 + be0), 0.0)
          m1 = jnp.maximum(x_v[pl.ds(s + L, L)] + (a * we1 + be1), 0.0)
          plsc.addupdate(aggr_v.at[pl.ds(d, L)], m0)
          plsc.addupdate(aggr_v.at[pl.ds(d + L, L)], m1)

      pltpu.sync_copy(aggr_v, out_hbm.at[g, q])
      return carry

    lax.fori_loop(0, TPW, task, 0)

  return k(x, src, dst, ea, params)


GB = 8


def _mlp_body(x_ref, p_ref, w1_ref, b1_ref, w2_ref, b2_ref, o_ref):
  xb = x_ref[...]
  h = xb + p_ref[...].sum(axis=1)
  h2 = jnp.maximum(
      h.reshape(GB * N, IN_C) @ w1_ref[...] + b1_ref[...], 0.0)
  pooled = h2.reshape(GB, N, HID).sum(axis=1) * (1.0 / N)
  o_ref[...] = pooled @ w2_ref[...] + b2_ref[...]


def _mlp(x, partials, W1, b1, W2, b2):
  return pl.pallas_call(
      _mlp_body,
      out_shape=jax.ShapeDtypeStruct((G, OUT), jnp.float32),
      grid=(G // GB,),
      in_specs=[
          pl.BlockSpec((GB, N, IN_C), lambda i: (i, 0, 0)),
          pl.BlockSpec((GB, Q, N, IN_C), lambda i: (i, 0, 0, 0)),
          pl.BlockSpec((IN_C, HID), lambda i: (0, 0)),
          pl.BlockSpec((1, HID), lambda i: (0, 0)),
          pl.BlockSpec((HID, OUT), lambda i: (0, 0)),
          pl.BlockSpec((1, OUT), lambda i: (0, 0)),
      ],
      out_specs=pl.BlockSpec((GB, OUT), lambda i: (i, 0)),
  )(x, partials, W1, b1, W2, b2)


def kernel(x, edge_index, edge_attr, W_edge, b_edge, W1, b1, W2, b2):
  x3 = x.reshape(G, N, IN_C)
  ei = edge_index.reshape(G, 2, E).astype(jnp.int32)
  src = ei[:, 0, :]
  dst = ei[:, 1, :]
  ea = edge_attr.reshape(G, E)
  params = jnp.concatenate(
      [W_edge.reshape(-1), b_edge.reshape(-1)]).astype(jnp.float32)
  partials = _sc_scatter(x3.reshape(G, NC), src, dst, ea, params)
  tokens = _mlp(x3, partials.reshape(G, Q, N, IN_C), W1,
                b1.reshape(1, HID), W2, b2.reshape(1, OUT))
  return tokens.reshape(B, BANDS, OUT)
